# baseline (speedup 1.0000x reference)
import functools

import jax
import jax.numpy as jnp
from jax import lax
from jax.experimental import pallas as pl
from jax.experimental.pallas import tpu as pltpu

SX, SY, SZ = 128, 128, 16
X2, Y2 = SX + 2, SY + 2
NROWS = X2 * Y2 * SZ
R = 2704
NBLK = NROWS // R
C = 64
NEG_SLOPE = 0.01
EPS = 1e-5

_OFFSETS = [(dx, dy, dz)
            for dx in (-1, 0, 1) for dy in (-1, 0, 1) for dz in (-1, 0, 1)]


def _zmask(jz, dz, a):
    zero = jnp.zeros((), a.dtype)
    if dz == 1:
        return jnp.where(jz != SZ - 1, a, zero)
    if dz == -1:
        return jnp.where(jz != 0, a, zero)
    return a


def _accumulate(w_ref, scratch):
    jz = lax.broadcasted_iota(jnp.int32, (R, 1), 0) % SZ
    slices = []
    for dx, dy, dz in _OFFSETS:
        d = (dx * Y2 + dy) * SZ + dz
        a = scratch[pl.ds(R + d, R), :]
        slices.append(_zmask(jz, dz, a))
    a27 = jnp.concatenate(slices, axis=1)
    return jnp.dot(a27, w_ref[...], preferred_element_type=jnp.float32)


def _stats(o_ref, s1_ref, s2_ref, mult_ref, acc):
    o_ref[...] = acc.astype(jnp.bfloat16)
    m = mult_ref[...]
    s1_ref[...] = jnp.sum(acc * m, axis=0, keepdims=True)[None]
    s2_ref[...] = jnp.sum(acc * acc * m, axis=0, keepdims=True)[None]


def _conv1_body(w_ref, mult_ref, prev_ref, cur_ref, nxt_ref,
                o_ref, s1_ref, s2_ref, scratch):
    scratch[pl.ds(0, R), :] = prev_ref[...]
    scratch[pl.ds(R, R), :] = cur_ref[...]
    scratch[pl.ds(2 * R, R), :] = nxt_ref[...]
    acc = _accumulate(w_ref, scratch)
    _stats(o_ref, s1_ref, s2_ref, mult_ref, acc)


def _conv2_body(w_ref, mult_ref, mu_ref, inv_ref, g_ref, b_ref,
                prev_ref, cur_ref, nxt_ref,
                oprev_ref, ocur_ref, onxt_ref,
                o_ref, s1_ref, s2_ref, scratch):
    def prep(x, occ):
        y = (x.astype(jnp.float32) - mu_ref[...]) * inv_ref[...] * g_ref[...] \
            + b_ref[...]
        y = jnp.where(y >= 0, y, NEG_SLOPE * y)
        return (y * occ).astype(jnp.bfloat16)

    scratch[pl.ds(0, R), :] = prep(prev_ref[...], oprev_ref[...])
    scratch[pl.ds(R, R), :] = prep(cur_ref[...], ocur_ref[...])
    scratch[pl.ds(2 * R, R), :] = prep(nxt_ref[...], onxt_ref[...])
    acc = _accumulate(w_ref, scratch)
    _stats(o_ref, s1_ref, s2_ref, mult_ref, acc)


_ROWSPEC = dict(
    prev=pl.BlockSpec((R, C), lambda i: (jnp.maximum(i - 1, 0), 0)),
    cur=pl.BlockSpec((R, C), lambda i: (i, 0)),
    nxt=pl.BlockSpec((R, C), lambda i: (jnp.minimum(i + 1, NBLK - 1), 0)),
)


def _conv1(grid_feat, mult, w):
    return pl.pallas_call(
        _conv1_body,
        grid=(NBLK,),
        in_specs=[
            pl.BlockSpec((27 * C, C), lambda i: (0, 0)),
            pl.BlockSpec((R, 1), lambda i: (i, 0)),
            _ROWSPEC["prev"], _ROWSPEC["cur"], _ROWSPEC["nxt"],
        ],
        out_specs=[
            pl.BlockSpec((R, C), lambda i: (i, 0)),
            pl.BlockSpec((1, 1, C), lambda i: (i, 0, 0)),
            pl.BlockSpec((1, 1, C), lambda i: (i, 0, 0)),
        ],
        out_shape=[
            jax.ShapeDtypeStruct((NROWS, C), jnp.bfloat16),
            jax.ShapeDtypeStruct((NBLK, 1, C), jnp.float32),
            jax.ShapeDtypeStruct((NBLK, 1, C), jnp.float32),
        ],
        scratch_shapes=[pltpu.VMEM((3 * R, C), jnp.bfloat16)],
        compiler_params=pltpu.CompilerParams(
            dimension_semantics=("arbitrary",)),
    )(w, mult, grid_feat, grid_feat, grid_feat)


def _conv2(grid_feat, occ, mult, w, mu, inv, g, b):
    return pl.pallas_call(
        _conv2_body,
        grid=(NBLK,),
        in_specs=[
            pl.BlockSpec((27 * C, C), lambda i: (0, 0)),
            pl.BlockSpec((R, 1), lambda i: (i, 0)),
            pl.BlockSpec((1, C), lambda i: (0, 0)),
            pl.BlockSpec((1, C), lambda i: (0, 0)),
            pl.BlockSpec((1, C), lambda i: (0, 0)),
            pl.BlockSpec((1, C), lambda i: (0, 0)),
            _ROWSPEC["prev"], _ROWSPEC["cur"], _ROWSPEC["nxt"],
            pl.BlockSpec((R, 1), lambda i: (jnp.maximum(i - 1, 0), 0)),
            pl.BlockSpec((R, 1), lambda i: (i, 0)),
            pl.BlockSpec((R, 1), lambda i: (jnp.minimum(i + 1, NBLK - 1), 0)),
        ],
        out_specs=[
            pl.BlockSpec((R, C), lambda i: (i, 0)),
            pl.BlockSpec((1, 1, C), lambda i: (i, 0, 0)),
            pl.BlockSpec((1, 1, C), lambda i: (i, 0, 0)),
        ],
        out_shape=[
            jax.ShapeDtypeStruct((NROWS, C), jnp.bfloat16),
            jax.ShapeDtypeStruct((NBLK, 1, C), jnp.float32),
            jax.ShapeDtypeStruct((NBLK, 1, C), jnp.float32),
        ],
        scratch_shapes=[pltpu.VMEM((3 * R, C), jnp.bfloat16)],
        compiler_params=pltpu.CompilerParams(
            dimension_semantics=("arbitrary",)),
    )(w, mult, mu, inv, g, b, grid_feat, grid_feat, grid_feat, occ, occ, occ)


def _final_body(t_ref, v_ref, mu_ref, inv_ref, g_ref, b_ref, o_ref):
    t = t_ref[...].astype(jnp.float32)
    y = (t - mu_ref[...]) * inv_ref[...] * g_ref[...] + b_ref[...]
    y = y + v_ref[...]
    o_ref[...] = jnp.where(y >= 0, y, NEG_SLOPE * y)


def _final(t, v_fea, mu, inv, g, b):
    n = t.shape[0]
    blk = 8000
    vspec = pl.BlockSpec((1, C), lambda i: (0, 0))
    return pl.pallas_call(
        _final_body,
        grid=(n // blk,),
        in_specs=[pl.BlockSpec((blk, C), lambda i: (i, 0)),
                  pl.BlockSpec((blk, C), lambda i: (i, 0)),
                  vspec, vspec, vspec, vspec],
        out_specs=pl.BlockSpec((blk, C), lambda i: (i, 0)),
        out_shape=jax.ShapeDtypeStruct((n, C), jnp.float32),
    )(t, v_fea, mu, inv, g, b)


def _finalize_stats(s1, s2, n_v):
    mu = jnp.sum(s1[:, 0, :], axis=0, keepdims=True) / n_v
    ex2 = jnp.sum(s2[:, 0, :], axis=0, keepdims=True) / n_v
    var = ex2 - mu * mu
    inv = 1.0 / jnp.sqrt(var + EPS)
    return mu, inv


def kernel(features, coors_inv_last, coors_inv, coors, W1, g1, b1, W2, g2, b2):
    n_v = coors.shape[0]
    n_pts = coors_inv.shape[0]

    gathered = features[coors_inv_last]
    sums = jax.ops.segment_sum(gathered, coors_inv, num_segments=n_v)
    cnt = jax.ops.segment_sum(jnp.ones((n_pts, 1), jnp.float32),
                              coors_inv, num_segments=n_v)
    v_fea = sums / jnp.maximum(cnt, 1.0)

    cx = coors[:, 1].astype(jnp.int32)
    cy = coors[:, 2].astype(jnp.int32)
    cz = coors[:, 3].astype(jnp.int32)
    lin = cx * (SY * SZ) + cy * SZ + cz
    grid_idx = jnp.full((SX * SY * SZ,), -1, jnp.int32).at[lin].set(
        jnp.arange(n_v, dtype=jnp.int32))
    cnt_cell = jnp.zeros((SX * SY * SZ,), jnp.float32).at[lin].add(1.0)

    idx_pad = jnp.pad(grid_idx.reshape(SX, SY, SZ),
                      ((1, 1), (1, 1), (0, 0)), constant_values=-1).reshape(-1)
    mult = jnp.pad(cnt_cell.reshape(SX, SY, SZ),
                   ((1, 1), (1, 1), (0, 0))).reshape(-1, 1)
    occ = (idx_pad >= 0).astype(jnp.float32)[:, None]

    safe = jnp.where(idx_pad < 0, n_v, idx_pad)
    v_ext = jnp.concatenate([v_fea.astype(jnp.bfloat16),
                             jnp.zeros((1, C), jnp.bfloat16)], axis=0)
    grid_feat = v_ext[safe]

    w1 = W1.reshape(27 * C, C).astype(jnp.bfloat16)
    w2 = W2.reshape(27 * C, C).astype(jnp.bfloat16)

    o1, s1a, s1b = _conv1(grid_feat, mult, w1)
    mu1, inv1 = _finalize_stats(s1a, s1b, float(n_v))

    o2, s2a, s2b = _conv2(o1, occ, mult, w2, mu1, inv1,
                          g1[None, :], b1[None, :])
    mu2, inv2 = _finalize_stats(s2a, s2b, float(n_v))

    lin_pad = ((cx + 1) * Y2 + (cy + 1)) * SZ + cz
    t = o2[lin_pad]

    return _final(t, v_fea, mu2, inv2, g2[None, :], b2[None, :])


# speedup vs baseline: 1.4254x; 1.4254x over previous
import functools

import jax
import jax.numpy as jnp
from jax import lax
from jax.experimental import pallas as pl
from jax.experimental.pallas import tpu as pltpu
from jax.experimental.pallas import tpu_sc as plsc

SX, SY, SZ = 128, 128, 16
X2, Y2 = SX + 2, SY + 2
NROWS = X2 * Y2 * SZ
R = 2704
NBLK = NROWS // R
C = 64
NEG_SLOPE = 0.01
EPS = 1e-5

_OFFSETS = [(dx, dy, dz)
            for dx in (-1, 0, 1) for dy in (-1, 0, 1) for dz in (-1, 0, 1)]


def _zmask(jz, dz, a):
    zero = jnp.zeros((), a.dtype)
    if dz == 1:
        return jnp.where(jz != SZ - 1, a, zero)
    if dz == -1:
        return jnp.where(jz != 0, a, zero)
    return a


def _accumulate(w_ref, scratch):
    jz = lax.broadcasted_iota(jnp.int32, (R, 1), 0) % SZ
    slices = []
    for dx, dy, dz in _OFFSETS:
        d = (dx * Y2 + dy) * SZ + dz
        a = scratch[pl.ds(R + d, R), :]
        slices.append(_zmask(jz, dz, a))
    a27 = jnp.concatenate(slices, axis=1)
    return jnp.dot(a27, w_ref[...], preferred_element_type=jnp.float32)


def _stats(o_ref, s1_ref, s2_ref, mult_ref, acc):
    o_ref[...] = acc
    m = mult_ref[...]
    s1_ref[...] = jnp.sum(acc * m, axis=0, keepdims=True)[None]
    s2_ref[...] = jnp.sum(acc * acc * m, axis=0, keepdims=True)[None]


def _conv1_body(w_ref, mult_ref, prev_ref, cur_ref, nxt_ref,
                o_ref, s1_ref, s2_ref, scratch):
    scratch[pl.ds(0, R), :] = prev_ref[...]
    scratch[pl.ds(R, R), :] = cur_ref[...]
    scratch[pl.ds(2 * R, R), :] = nxt_ref[...]
    acc = _accumulate(w_ref, scratch)
    _stats(o_ref, s1_ref, s2_ref, mult_ref, acc)


def _conv2_body(w_ref, mult_ref, mu_ref, inv_ref, g_ref, b_ref,
                prev_ref, cur_ref, nxt_ref,
                oprev_ref, ocur_ref, onxt_ref,
                o_ref, s1_ref, s2_ref, scratch):
    def prep(x, occ):
        y = (x - mu_ref[...]) * inv_ref[...] * g_ref[...] + b_ref[...]
        y = jnp.where(y >= 0, y, NEG_SLOPE * y)
        return y * occ

    scratch[pl.ds(0, R), :] = prep(prev_ref[...], oprev_ref[...])
    scratch[pl.ds(R, R), :] = prep(cur_ref[...], ocur_ref[...])
    scratch[pl.ds(2 * R, R), :] = prep(nxt_ref[...], onxt_ref[...])
    acc = _accumulate(w_ref, scratch)
    _stats(o_ref, s1_ref, s2_ref, mult_ref, acc)


_ROWSPEC = dict(
    prev=pl.BlockSpec((R, C), lambda i: (jnp.maximum(i - 1, 0), 0)),
    cur=pl.BlockSpec((R, C), lambda i: (i, 0)),
    nxt=pl.BlockSpec((R, C), lambda i: (jnp.minimum(i + 1, NBLK - 1), 0)),
)


def _conv1(grid_feat, mult, w):
    return pl.pallas_call(
        _conv1_body,
        grid=(NBLK,),
        in_specs=[
            pl.BlockSpec((27 * C, C), lambda i: (0, 0)),
            pl.BlockSpec((R, 1), lambda i: (i, 0)),
            _ROWSPEC["prev"], _ROWSPEC["cur"], _ROWSPEC["nxt"],
        ],
        out_specs=[
            pl.BlockSpec((R, C), lambda i: (i, 0)),
            pl.BlockSpec((1, 1, C), lambda i: (i, 0, 0)),
            pl.BlockSpec((1, 1, C), lambda i: (i, 0, 0)),
        ],
        out_shape=[
            jax.ShapeDtypeStruct((NROWS, C), jnp.float32),
            jax.ShapeDtypeStruct((NBLK, 1, C), jnp.float32),
            jax.ShapeDtypeStruct((NBLK, 1, C), jnp.float32),
        ],
        scratch_shapes=[pltpu.VMEM((3 * R, C), jnp.float32)],
        compiler_params=pltpu.CompilerParams(
            dimension_semantics=("arbitrary",)),
    )(w, mult, grid_feat, grid_feat, grid_feat)


def _conv2(grid_feat, occ, mult, w, mu, inv, g, b):
    return pl.pallas_call(
        _conv2_body,
        grid=(NBLK,),
        in_specs=[
            pl.BlockSpec((27 * C, C), lambda i: (0, 0)),
            pl.BlockSpec((R, 1), lambda i: (i, 0)),
            pl.BlockSpec((1, C), lambda i: (0, 0)),
            pl.BlockSpec((1, C), lambda i: (0, 0)),
            pl.BlockSpec((1, C), lambda i: (0, 0)),
            pl.BlockSpec((1, C), lambda i: (0, 0)),
            _ROWSPEC["prev"], _ROWSPEC["cur"], _ROWSPEC["nxt"],
            pl.BlockSpec((R, 1), lambda i: (jnp.maximum(i - 1, 0), 0)),
            pl.BlockSpec((R, 1), lambda i: (i, 0)),
            pl.BlockSpec((R, 1), lambda i: (jnp.minimum(i + 1, NBLK - 1), 0)),
        ],
        out_specs=[
            pl.BlockSpec((R, C), lambda i: (i, 0)),
            pl.BlockSpec((1, 1, C), lambda i: (i, 0, 0)),
            pl.BlockSpec((1, 1, C), lambda i: (i, 0, 0)),
        ],
        out_shape=[
            jax.ShapeDtypeStruct((NROWS, C), jnp.float32),
            jax.ShapeDtypeStruct((NBLK, 1, C), jnp.float32),
            jax.ShapeDtypeStruct((NBLK, 1, C), jnp.float32),
        ],
        scratch_shapes=[pltpu.VMEM((3 * R, C), jnp.float32)],
        compiler_params=pltpu.CompilerParams(
            dimension_semantics=("arbitrary",)),
    )(w, mult, mu, inv, g, b, grid_feat, grid_feat, grid_feat, occ, occ, occ)


def _final_body(t_ref, v_ref, mu_ref, inv_ref, g_ref, b_ref, o_ref):
    t = t_ref[...]
    y = (t - mu_ref[...]) * inv_ref[...] * g_ref[...] + b_ref[...]
    y = y + v_ref[...]
    o_ref[...] = jnp.where(y >= 0, y, NEG_SLOPE * y)


def _final(t, v_fea, mu, inv, g, b):
    n = t.shape[0]
    blk = 8000
    vspec = pl.BlockSpec((1, C), lambda i: (0, 0))
    return pl.pallas_call(
        _final_body,
        grid=(n // blk,),
        in_specs=[pl.BlockSpec((blk, C), lambda i: (i, 0)),
                  pl.BlockSpec((blk, C), lambda i: (i, 0)),
                  vspec, vspec, vspec, vspec],
        out_specs=pl.BlockSpec((blk, C), lambda i: (i, 0)),
        out_shape=jax.ShapeDtypeStruct((n, C), jnp.float32),
    )(t, v_fea, mu, inv, g, b)


def _finalize_stats(s1, s2, n_v):
    mu = jnp.sum(s1[:, 0, :], axis=0, keepdims=True) / n_v
    ex2 = jnp.sum(s2[:, 0, :], axis=0, keepdims=True) / n_v
    var = ex2 - mu * mu
    inv = 1.0 / jnp.sqrt(var + EPS)
    return mu, inv


_VC = 10000
_PT = 20000
_PTP = 20480
_B = 128
_NBAT = _PTP // _B
_AROWS = 10112


def _sc_body(feat, civh, cilh, zfh, zch, onesh, sums, cnto,
             civ_v, cil_v, stage, sidx, ones_v, accum, cacc,
             gs0, gs1, gs2, gs3):
    c = lax.axis_index("c")
    t = lax.axis_index("s")
    gsems = (gs0, gs1, gs2, gs3)
    i16 = lax.iota(jnp.int32, 16)

    pltpu.sync_copy(civh.at[pl.ds(t * _PT, _PT)], civ_v.at[pl.ds(0, _PT)])
    pltpu.sync_copy(cilh.at[pl.ds(t * _PT, _PT)], cil_v.at[pl.ds(0, _PT)])
    for j in range(_PT, _PTP, 16):
        civ_v[pl.ds(j, 16)] = jnp.full((16,), -1, jnp.int32)
        cil_v[pl.ds(j, 16)] = jnp.zeros((16,), jnp.int32)
    pltpu.sync_copy(onesh, ones_v)

    def fire_gather(b, slot):
        pltpu.make_async_copy(
            feat.at[cil_v.at[pl.ds(b * _B, _B)]],
            stage.at[slot], gsems[slot]).start()

    def drain_gather(slot):
        pltpu.make_async_copy(
            feat.at[cil_v.at[pl.ds(0, _B)]],
            stage.at[slot], gsems[slot]).wait()

    for p in range(4):
        chunk = c * 4 + p
        base = chunk * _VC
        pltpu.sync_copy(zfh, accum.at[pl.ds(t * 632, 632)])
        pltpu.sync_copy(zch, cacc.at[pl.ds(t * 632, 632)])
        plsc.subcore_barrier()

        for s in range(4):
            fire_gather(s, s)

        def unit(b, slot):
            drain_gather(slot)
            srow = sidx.at[slot]
            for j in range(8):
                civ16 = civ_v[pl.ds(b * _B + j * 16, 16)]
                m = (civ16 >= base) & (civ16 < base + _VC)
                l = civ16 - base
                srow[pl.ds(j * 16, 16)] = jnp.where(m, l, _VC + i16)
            pltpu.sync_copy(stage.at[slot], accum.at[srow], add=True)
            pltpu.sync_copy(ones_v, cacc.at[srow], add=True)

            @pl.when(b + 4 < _NBAT)
            def _():
                fire_gather(b + 4, slot)

        def outer(g, carry):
            for s in range(4):
                unit(g * 4 + s, s)
            return carry

        lax.fori_loop(0, _NBAT // 4, outer, 0)
        plsc.subcore_barrier()

        @pl.when(t < 15)
        def _():
            pltpu.sync_copy(accum.at[pl.ds(t * 624, 624)],
                            sums.at[pl.ds(base + t * 624, 624)])
            pltpu.sync_copy(cacc.at[pl.ds(t * 624, 624)],
                            cnto.at[chunk, pl.ds(t * 624, 624)])

        @pl.when(t == 15)
        def _():
            pltpu.sync_copy(accum.at[pl.ds(9360, 640)],
                            sums.at[pl.ds(base + 9360, 640)])
            pltpu.sync_copy(cacc.at[pl.ds(9360, 640)],
                            cnto.at[chunk, pl.ds(9360, 640)])

        plsc.subcore_barrier()


def _scatter_mean_sc(features, civ, cil):
    zf = jnp.zeros((632, 64), jnp.float32)
    zc = jnp.zeros((632, 16), jnp.float32)
    ones = jnp.ones((_B, 16), jnp.float32)
    mesh = plsc.VectorSubcoreMesh(core_axis_name="c", subcore_axis_name="s")
    fn = functools.partial(
        pl.kernel, mesh=mesh,
        compiler_params=pltpu.CompilerParams(use_tc_tiling_on_sc=False),
        out_type=[jax.ShapeDtypeStruct((80000, 64), jnp.float32),
                  jax.ShapeDtypeStruct((8, _VC, 16), jnp.float32)],
        scratch_types=[
            pltpu.VMEM((_PTP,), jnp.int32),
            pltpu.VMEM((_PTP,), jnp.int32),
            pltpu.VMEM((4, _B, 64), jnp.float32),
            pltpu.VMEM((4, _B), jnp.int32),
            pltpu.VMEM((_B, 16), jnp.float32),
            pltpu.VMEM_SHARED((_AROWS, 64), jnp.float32),
            pltpu.VMEM_SHARED((_AROWS, 16), jnp.float32),
            pltpu.SemaphoreType.DMA,
            pltpu.SemaphoreType.DMA,
            pltpu.SemaphoreType.DMA,
            pltpu.SemaphoreType.DMA,
        ],
    )(_sc_body)
    sums, cnt3d = fn(features, civ, cil, zf, zc, ones)
    cnt = cnt3d[:, :, 0].reshape(80000, 1)
    return sums, cnt


def _div_body(s_ref, c_ref, o_ref):
    o_ref[...] = s_ref[...] / jnp.maximum(c_ref[...], 1.0)


def _divide(sums, cnt):
    n = sums.shape[0]
    blk = 8000
    return pl.pallas_call(
        _div_body,
        grid=(n // blk,),
        in_specs=[pl.BlockSpec((blk, C), lambda i: (i, 0)),
                  pl.BlockSpec((blk, 1), lambda i: (i, 0))],
        out_specs=pl.BlockSpec((blk, C), lambda i: (i, 0)),
        out_shape=jax.ShapeDtypeStruct((n, C), jnp.float32),
    )(sums, cnt)


def kernel(features, coors_inv_last, coors_inv, coors, W1, g1, b1, W2, g2, b2):
    n_v = coors.shape[0]
    n_pts = coors_inv.shape[0]

    civ = coors_inv.astype(jnp.int32)
    cil = coors_inv_last.astype(jnp.int32)
    sums, cnt = _scatter_mean_sc(features, civ, cil)
    v_fea = _divide(sums, cnt)

    cx = coors[:, 1].astype(jnp.int32)
    cy = coors[:, 2].astype(jnp.int32)
    cz = coors[:, 3].astype(jnp.int32)
    lin = cx * (SY * SZ) + cy * SZ + cz
    grid_idx = jnp.full((SX * SY * SZ,), -1, jnp.int32).at[lin].set(
        jnp.arange(n_v, dtype=jnp.int32))
    cnt_cell = jnp.zeros((SX * SY * SZ,), jnp.float32).at[lin].add(1.0)

    idx_pad = jnp.pad(grid_idx.reshape(SX, SY, SZ),
                      ((1, 1), (1, 1), (0, 0)), constant_values=-1).reshape(-1)
    mult = jnp.pad(cnt_cell.reshape(SX, SY, SZ),
                   ((1, 1), (1, 1), (0, 0))).reshape(-1, 1)
    occ = (idx_pad >= 0).astype(jnp.float32)[:, None]

    safe = jnp.where(idx_pad < 0, n_v, idx_pad)
    v_ext = jnp.concatenate([v_fea, jnp.zeros((1, C), jnp.float32)], axis=0)
    grid_feat = v_ext[safe]

    w1 = W1.reshape(27 * C, C)
    w2 = W2.reshape(27 * C, C)

    o1, s1a, s1b = _conv1(grid_feat, mult, w1)
    mu1, inv1 = _finalize_stats(s1a, s1b, float(n_v))

    o2, s2a, s2b = _conv2(o1, occ, mult, w2, mu1, inv1,
                          g1[None, :], b1[None, :])
    mu2, inv2 = _finalize_stats(s2a, s2b, float(n_v))

    lin_pad = ((cx + 1) * Y2 + (cy + 1)) * SZ + cz
    t = o2[lin_pad]

    return _final(t, v_fea, mu2, inv2, g2[None, :], b2[None, :])
